# baseline (speedup 1.0000x reference)
import functools

import jax
import jax.numpy as jnp
from jax import lax
from jax.experimental import pallas as pl
from jax.experimental.pallas import tpu as pltpu
from jax.experimental.pallas import tpu_sc as plsc

_N_FIELDS = 26
_VOCAB = 100000
_DIM = 16
_MAX_NORM = 4.0
_BATCH = 4096
_TIME = 20

_NROWS = _BATCH * _TIME * _N_FIELDS
_IDX_ROWS = _NROWS // 128

_NC = 2
_NS = 16
_NW = _NC * _NS
_G = _IDX_ROWS // _NW
_K = 8
_CHUNK_ROWS = _K * 128
_PACKED_ROWS = _NROWS // 8
_NROWS_TBL = _N_FIELDS * _VOCAB


@functools.partial(
    pl.kernel,
    mesh=plsc.VectorSubcoreMesh(core_axis_name="c", subcore_axis_name="s"),
    out_type=jax.ShapeDtypeStruct((_NROWS, _DIM), jnp.float32),
    scratch_types=[
        pltpu.VMEM((_K, 128), jnp.int32),
        pltpu.VMEM((_CHUNK_ROWS, _DIM), jnp.float32),
        pltpu.SemaphoreType.DMA,
    ],
    compiler_params=pltpu.CompilerParams(use_tc_tiling_on_sc=False),
)
def _sc_gather(table_hbm, idx_hbm, out_hbm, idx_v, rows_v, sem):
    wid = lax.axis_index("s") * _NC + lax.axis_index("c")
    base = wid * _G
    def chunk(c, carry):
        r0 = base + c * _K
        pltpu.sync_copy(idx_hbm.at[pl.ds(r0, _K)], idx_v)
        copies = [
            pltpu.async_copy(
                table_hbm.at[idx_v.at[j]],
                rows_v.at[pl.ds(j * 128, 128)],
                sem,
            )
            for j in range(_K)
        ]
        for cp in copies:
            cp.wait()
        pltpu.sync_copy(rows_v, out_hbm.at[pl.ds(r0 * 128, _CHUNK_ROWS)])
        return carry

    lax.fori_loop(0, _G // _K, chunk, 0)


_TC_BLK = 1024


def _tc_body(v_ref, o_ref):
    v = v_ref[...]
    v2 = v * v
    li = lax.broadcasted_iota(jnp.int32, (128, 128), 0) // _DIM
    lj = lax.broadcasted_iota(jnp.int32, (128, 128), 1) // _DIM
    m = (li == lj).astype(jnp.float32)
    ss = lax.dot_general(
        v2, m, (((1,), (0,)), ((), ())), preferred_element_type=jnp.float32
    )
    scale = jnp.where(
        ss > _MAX_NORM * _MAX_NORM, _MAX_NORM * lax.rsqrt(ss), 1.0
    )
    v = v * scale
    u = 1.0 + jnp.exp(v)
    o_ref[...] = v * (1.0 - 2.0 / (u * u + 1.0))


_tc_post = pl.pallas_call(
    _tc_body,
    grid=(_PACKED_ROWS // _TC_BLK,),
    in_specs=[pl.BlockSpec((_TC_BLK, 128), lambda i: (i, 0))],
    out_specs=pl.BlockSpec((_TC_BLK, 128), lambda i: (i, 0)),
    out_shape=jax.ShapeDtypeStruct((_PACKED_ROWS, 128), jnp.float32),
)


def kernel(x, emb):
    idx = x.reshape(-1, _N_FIELDS) + (
        jnp.arange(_N_FIELDS, dtype=jnp.int32) * _VOCAB
    )
    idx = idx.reshape(_IDX_ROWS, 128)
    t1 = lax.optimization_barrier(emb.reshape(_NROWS_TBL * _DIM // 128, 128))
    table = t1.reshape(_NROWS_TBL, _DIM)
    rows = _sc_gather(table, idx)
    out = _tc_post(rows.reshape(_PACKED_ROWS, 128))
    return out.reshape(_BATCH, _TIME, _N_FIELDS * _DIM)


# speedup vs baseline: 1.2432x; 1.2432x over previous
import functools

import jax
import jax.numpy as jnp
from jax import lax
from jax.experimental import pallas as pl
from jax.experimental.pallas import tpu as pltpu
from jax.experimental.pallas import tpu_sc as plsc

_N_FIELDS = 26
_VOCAB = 100000
_DIM = 16
_MAX_NORM = 4.0
_BATCH = 4096
_TIME = 20

_NROWS = _BATCH * _TIME * _N_FIELDS
_IDX_ROWS = _NROWS // 128

_NC = 2
_NS = 16
_NW = _NC * _NS
_G = _IDX_ROWS // _NW
_K = 8
_CHUNK_ROWS = _K * 128
_PACKED_ROWS = _NROWS // 8
_NROWS_TBL = _N_FIELDS * _VOCAB


@functools.partial(
    pl.kernel,
    mesh=plsc.VectorSubcoreMesh(core_axis_name="c", subcore_axis_name="s"),
    out_type=jax.ShapeDtypeStruct((_NROWS, _DIM), jnp.float32),
    scratch_types=[
        pltpu.VMEM((_K, 128), jnp.int32),
        pltpu.VMEM((_CHUNK_ROWS, _DIM), jnp.float32),
        pltpu.SemaphoreType.DMA,
    ],
    compiler_params=pltpu.CompilerParams(use_tc_tiling_on_sc=False),
)
def _sc_gather(table_hbm, idx_hbm, out_hbm, idx_v, rows_v, sem):
    wid = lax.axis_index("s") * _NC + lax.axis_index("c")
    base = wid * _G
    def chunk(c, carry):
        r0 = base + c * _K
        pltpu.sync_copy(idx_hbm.at[pl.ds(r0, _K)], idx_v)
        lane = lax.iota(jnp.int32, 16)
        for j in range(_K):
            rowbase = (r0 + j) * 128
            for l in range(8):
                p = lane + (rowbase + l * 16)
                f = lax.rem(p, jnp.int32(_N_FIELDS))
                sl = (j, pl.ds(l * 16, 16))
                idx_v[sl] = idx_v[sl] + f * jnp.int32(_VOCAB)
        copies = [
            pltpu.async_copy(
                table_hbm.at[idx_v.at[j]],
                rows_v.at[pl.ds(j * 128, 128)],
                sem,
            )
            for j in range(_K)
        ]
        for cp in copies:
            cp.wait()
        pltpu.sync_copy(rows_v, out_hbm.at[pl.ds(r0 * 128, _CHUNK_ROWS)])
        return carry

    lax.fori_loop(0, _G // _K, chunk, 0)


_TC_BLK = 1024


_ROWS_PER_B = _N_FIELDS * _DIM * _TIME // 128
_BBLK = 128
_TC_ROWS = _ROWS_PER_B * _BBLK


def _tc_body(v_ref, o_ref):
    v = v_ref[...]
    v2 = v * v
    li = lax.broadcasted_iota(jnp.int32, (128, 128), 0) // _DIM
    lj = lax.broadcasted_iota(jnp.int32, (128, 128), 1) // _DIM
    m = (li == lj).astype(jnp.float32)
    ss = lax.dot_general(
        v2, m, (((1,), (0,)), ((), ())), preferred_element_type=jnp.float32
    )
    scale = jnp.where(
        ss > _MAX_NORM * _MAX_NORM, _MAX_NORM * lax.rsqrt(ss), 1.0
    )
    v = v * scale
    u = 1.0 + jnp.exp(v)
    w = v * (1.0 - 2.0 / (u * u + 1.0))
    w3 = w.reshape(_BBLK, _ROWS_PER_B, 128)
    for s in range(_ROWS_PER_B):
        o_ref[pl.ds(s * 128, 128), :] = w3[:, s, :].T


_tc_post = pl.pallas_call(
    _tc_body,
    grid=(_BATCH // _BBLK,),
    in_specs=[pl.BlockSpec((_TC_ROWS, 128), lambda i: (i, 0))],
    out_specs=pl.BlockSpec((_TIME * _N_FIELDS * _DIM, _BBLK), lambda i: (0, i)),
    out_shape=jax.ShapeDtypeStruct((_TIME * _N_FIELDS * _DIM, _BATCH), jnp.float32),
)


def kernel(x, emb):
    idx = x.reshape(_IDX_ROWS, 128)
    t1 = lax.optimization_barrier(emb.reshape(_NROWS_TBL * _DIM // 128, 128))
    table = t1.reshape(_NROWS_TBL, _DIM)
    rows = _sc_gather(table, idx)
    out = _tc_post(rows.reshape(_PACKED_ROWS, 128))
    out = out.reshape(_TIME, _N_FIELDS * _DIM, _BATCH)
    return jnp.transpose(out, (2, 0, 1))
